# Initial kernel scaffold; baseline (speedup 1.0000x reference)
import jax
import jax.numpy as jnp
from jax.experimental import pallas as pl


def kernel(coords, boxsize, senders, receivers):
    raise NotImplementedError("write your pallas kernel here")


# speedup vs baseline: 11.4705x; 11.4705x over previous
import functools

import jax
import jax.numpy as jnp
from jax import lax
from jax.experimental import pallas as pl
from jax.experimental.pallas import tpu as pltpu
from jax.experimental.pallas import tpu_sc as plsc

_NC = 2
_NS = 16
_NW = _NC * _NS

_L = 16
_ROW = 128
_CR = 8
_CHUNK = _CR * _ROW

_RSQRT_MAGIC = 0x5F3759DF
_RNE_MAGIC = 12582912.0


def _sc_body(nchunks, trips, s_hbm, r_hbm, tab_hbm, box_hbm, out_r, out_rx,
             sidx_v, ridx_v, a_v, b_v, rx_v, rr_v, box_v, sem):
    wid = lax.axis_index("s") * _NC + lax.axis_index("c")

    pltpu.sync_copy(box_hbm, box_v)
    bx = box_v[0, :]
    by = box_v[1, :]
    bz = box_v[2, :]
    ibx = box_v[3, :]
    iby = box_v[4, :]
    ibz = box_v[5, :]

    iota = lax.iota(jnp.int32, _L)
    k0 = jnp.zeros((_L,), jnp.int32)
    k1 = jnp.full((_L,), 1, jnp.int32)
    k2 = jnp.full((_L,), 2, jnp.int32)
    mrne = jnp.full((_L,), _RNE_MAGIC, jnp.float32)
    half = jnp.full((_L,), 0.5, jnp.float32)
    c15 = jnp.full((_L,), 1.5, jnp.float32)
    magic = jnp.full((_L,), _RSQRT_MAGIC, jnp.int32)

    def chunk_body(t, carry):
        c = wid + _NW * t

        @pl.when(c < nchunks)
        def _():
            row0 = _CR * c
            pltpu.sync_copy(s_hbm.at[pl.ds(row0, _CR)], sidx_v)
            pltpu.sync_copy(r_hbm.at[pl.ds(row0, _CR)], ridx_v)
            copies = []
            for j in range(_CR):
                copies.append(
                    pltpu.async_copy(tab_hbm.at[sidx_v.at[j]],
                                     a_v.at[pl.ds(j * _ROW, _ROW)], sem))
                copies.append(
                    pltpu.async_copy(tab_hbm.at[ridx_v.at[j]],
                                     b_v.at[pl.ds(j * _ROW, _ROW)], sem))
            for cp in copies:
                cp.wait()

            def step(i, carry2):
                e16 = i * _L + iota
                ax = plsc.load_gather(a_v, [e16, k0])
                ay = plsc.load_gather(a_v, [e16, k1])
                az = plsc.load_gather(a_v, [e16, k2])
                qx = plsc.load_gather(b_v, [e16, k0])
                qy = plsc.load_gather(b_v, [e16, k1])
                qz = plsc.load_gather(b_v, [e16, k2])
                dx = qx - ax
                dy = qy - ay
                dz = qz - az
                nx = (dx * ibx + mrne) - mrne
                ny = (dy * iby + mrne) - mrne
                nz = (dz * ibz + mrne) - mrne
                dx = dx - nx * bx
                dy = dy - ny * by
                dz = dz - nz * bz
                s = dx * dx + dy * dy + dz * dz
                yi = magic - lax.shift_right_logical(plsc.bitcast(s, jnp.int32), 1)
                y = plsc.bitcast(yi, jnp.float32)
                hs = s * half
                y = y * (c15 - hs * y * y)
                y = y * (c15 - hs * y * y)
                y = y * (c15 - hs * y * y)
                rr = s * y
                plsc.store_scatter(rx_v, [e16, k0], dx)
                plsc.store_scatter(rx_v, [e16, k1], dy)
                plsc.store_scatter(rx_v, [e16, k2], dz)
                plsc.store_scatter(rr_v, [e16], rr)
                return carry2

            lax.fori_loop(0, _CHUNK // _L, step, 0, unroll=2)

            base = c * _CHUNK
            pltpu.sync_copy(rr_v, out_r.at[pl.ds(base, _CHUNK)])
            pltpu.sync_copy(rx_v, out_rx.at[pl.ds(base, _CHUNK)])

        return carry

    lax.fori_loop(0, trips, chunk_body, 0)


def kernel(coords, boxsize, senders, receivers):
    n_edges = senders.shape[0]
    assert n_edges % _CHUNK == 0
    nchunks = n_edges // _CHUNK
    trips = (nchunks + _NW - 1) // _NW

    s32 = senders.astype(jnp.int32).reshape(n_edges // _ROW, _ROW)
    r32 = receivers.astype(jnp.int32).reshape(n_edges // _ROW, _ROW)
    tab = jnp.pad(coords.astype(jnp.float32), ((0, 0), (0, 13)))
    box3 = boxsize.astype(jnp.float32).reshape(3)
    rows = [box3[0], box3[1], box3[2], 1.0 / box3[0], 1.0 / box3[1],
            1.0 / box3[2], jnp.float32(0.0), jnp.float32(0.0)]
    box_tab = jnp.stack([jnp.full((_L,), v, jnp.float32) for v in rows])

    mesh = plsc.VectorSubcoreMesh(core_axis_name="c", subcore_axis_name="s")
    f = functools.partial(
        pl.kernel,
        mesh=mesh,
        compiler_params=pltpu.CompilerParams(
            needs_layout_passes=False, use_tc_tiling_on_sc=False),
        out_type=[
            jax.ShapeDtypeStruct((n_edges,), jnp.float32),
            jax.ShapeDtypeStruct((n_edges, 3), jnp.float32),
        ],
        scratch_types=[
            pltpu.VMEM((_CR, _ROW), jnp.int32),
            pltpu.VMEM((_CR, _ROW), jnp.int32),
            pltpu.VMEM((_CHUNK, 16), jnp.float32),
            pltpu.VMEM((_CHUNK, 16), jnp.float32),
            pltpu.VMEM((_CHUNK, 3), jnp.float32),
            pltpu.VMEM((_CHUNK,), jnp.float32),
            pltpu.VMEM((8, _L), jnp.float32),
            pltpu.SemaphoreType.DMA,
        ],
    )(functools.partial(_sc_body, nchunks, trips))
    rr, rx = f(s32, r32, tab, box_tab)
    return (rr.reshape(n_edges, 1), rx)


# speedup vs baseline: 31.6047x; 2.7553x over previous
import functools

import jax
import jax.numpy as jnp
from jax import lax
from jax.experimental import pallas as pl
from jax.experimental.pallas import tpu as pltpu
from jax.experimental.pallas import tpu_sc as plsc

_NC = 2
_NS = 16
_NW = _NC * _NS

_L = 16
_ROW = 128
_CR = 8
_CHUNK = _CR * _ROW

_RSQRT_MAGIC = 0x5F3759DF
_RNE_MAGIC = 12582912.0


def _sc_body(nchunks, trips, s_hbm, r_hbm, tab_hbm, box_hbm,
             out_x, out_y, out_z, out_r,
             sidx_v, ridx_v, a_v, b_v, px_v, py_v, pz_v, rr_v, box_v, sem):
    wid = lax.axis_index("s") * _NC + lax.axis_index("c")

    pltpu.sync_copy(box_hbm, box_v)
    bx = box_v[0, :]
    by = box_v[1, :]
    bz = box_v[2, :]
    ibx = box_v[3, :]
    iby = box_v[4, :]
    ibz = box_v[5, :]

    iota = lax.iota(jnp.int32, _L)
    k0 = jnp.zeros((_L,), jnp.int32)
    k1 = jnp.full((_L,), 1, jnp.int32)
    k2 = jnp.full((_L,), 2, jnp.int32)
    mrne = jnp.full((_L,), _RNE_MAGIC, jnp.float32)
    half = jnp.full((_L,), 0.5, jnp.float32)
    c15 = jnp.full((_L,), 1.5, jnp.float32)
    magic = jnp.full((_L,), _RSQRT_MAGIC, jnp.int32)

    def chunk_body(t, carry):
        c = wid + _NW * t

        @pl.when(c < nchunks)
        def _():
            row0 = _CR * c
            pltpu.sync_copy(s_hbm.at[pl.ds(row0, _CR)], sidx_v)
            pltpu.sync_copy(r_hbm.at[pl.ds(row0, _CR)], ridx_v)
            copies = []
            for j in range(_CR):
                copies.append(
                    pltpu.async_copy(tab_hbm.at[sidx_v.at[j]],
                                     a_v.at[pl.ds(j * _ROW, _ROW)], sem))
                copies.append(
                    pltpu.async_copy(tab_hbm.at[ridx_v.at[j]],
                                     b_v.at[pl.ds(j * _ROW, _ROW)], sem))
            for cp in copies:
                cp.wait()

            def step(i, carry2):
                e16 = i * _L + iota
                ax = plsc.load_gather(a_v, [e16, k0])
                ay = plsc.load_gather(a_v, [e16, k1])
                az = plsc.load_gather(a_v, [e16, k2])
                qx = plsc.load_gather(b_v, [e16, k0])
                qy = plsc.load_gather(b_v, [e16, k1])
                qz = plsc.load_gather(b_v, [e16, k2])
                dx = qx - ax
                dy = qy - ay
                dz = qz - az
                nx = (dx * ibx + mrne) - mrne
                ny = (dy * iby + mrne) - mrne
                nz = (dz * ibz + mrne) - mrne
                dx = dx - nx * bx
                dy = dy - ny * by
                dz = dz - nz * bz
                s = dx * dx + dy * dy + dz * dz
                yi = magic - lax.shift_right_logical(plsc.bitcast(s, jnp.int32), 1)
                y = plsc.bitcast(yi, jnp.float32)
                hs = s * half
                y = y * (c15 - hs * y * y)
                y = y * (c15 - hs * y * y)
                y = y * (c15 - hs * y * y)
                rr = s * y
                sl = pl.ds(i * _L, _L)
                px_v[sl] = dx
                py_v[sl] = dy
                pz_v[sl] = dz
                rr_v[sl] = rr
                return carry2

            lax.fori_loop(0, _CHUNK // _L, step, 0, unroll=2)

            base = c * _CHUNK
            pltpu.sync_copy(px_v, out_x.at[pl.ds(base, _CHUNK)])
            pltpu.sync_copy(py_v, out_y.at[pl.ds(base, _CHUNK)])
            pltpu.sync_copy(pz_v, out_z.at[pl.ds(base, _CHUNK)])
            pltpu.sync_copy(rr_v, out_r.at[pl.ds(base, _CHUNK)])

        return carry

    lax.fori_loop(0, trips, chunk_body, 0)


def kernel(coords, boxsize, senders, receivers):
    n_edges = senders.shape[0]
    assert n_edges % _CHUNK == 0
    nchunks = n_edges // _CHUNK
    trips = (nchunks + _NW - 1) // _NW

    s32 = senders.astype(jnp.int32).reshape(n_edges // _ROW, _ROW)
    r32 = receivers.astype(jnp.int32).reshape(n_edges // _ROW, _ROW)
    tab = jnp.pad(coords.astype(jnp.float32), ((0, 0), (0, 13)))
    box3 = boxsize.astype(jnp.float32).reshape(3)
    rows = [box3[0], box3[1], box3[2], 1.0 / box3[0], 1.0 / box3[1],
            1.0 / box3[2], jnp.float32(0.0), jnp.float32(0.0)]
    box_tab = jnp.stack([jnp.full((_L,), v, jnp.float32) for v in rows])

    mesh = plsc.VectorSubcoreMesh(core_axis_name="c", subcore_axis_name="s")
    f = functools.partial(
        pl.kernel,
        mesh=mesh,
        compiler_params=pltpu.CompilerParams(
            needs_layout_passes=False, use_tc_tiling_on_sc=False),
        out_type=[
            jax.ShapeDtypeStruct((n_edges,), jnp.float32),
            jax.ShapeDtypeStruct((n_edges,), jnp.float32),
            jax.ShapeDtypeStruct((n_edges,), jnp.float32),
            jax.ShapeDtypeStruct((n_edges,), jnp.float32),
        ],
        scratch_types=[
            pltpu.VMEM((_CR, _ROW), jnp.int32),
            pltpu.VMEM((_CR, _ROW), jnp.int32),
            pltpu.VMEM((_CHUNK, 16), jnp.float32),
            pltpu.VMEM((_CHUNK, 16), jnp.float32),
            pltpu.VMEM((_CHUNK,), jnp.float32),
            pltpu.VMEM((_CHUNK,), jnp.float32),
            pltpu.VMEM((_CHUNK,), jnp.float32),
            pltpu.VMEM((_CHUNK,), jnp.float32),
            pltpu.VMEM((8, _L), jnp.float32),
            pltpu.SemaphoreType.DMA,
        ],
    )(functools.partial(_sc_body, nchunks, trips))
    px, py, pz, rr = f(s32, r32, tab, box_tab)
    rx = jnp.stack([px, py, pz], axis=1)
    return (rr.reshape(n_edges, 1), rx)


# speedup vs baseline: 47.4318x; 1.5008x over previous
import functools

import jax
import jax.numpy as jnp
from jax import lax
from jax.experimental import pallas as pl
from jax.experimental.pallas import tpu as pltpu
from jax.experimental.pallas import tpu_sc as plsc

_NC = 2
_NS = 16
_NW = _NC * _NS

_L = 16
_ROW = 128
_CR = 8
_CHUNK = _CR * _ROW

_RSQRT_MAGIC = 0x5F3759DF
_RNE_MAGIC = 12582912.0


def _sc_body(nchunks, trips, s_hbm, r_hbm, tab_hbm, box_hbm,
             out_x, out_y, out_z, out_r,
             sidx0, ridx0, a0, b0, sidx1, ridx1, a1, b1,
             px_v, py_v, pz_v, rr_v, box_v, sem0, sem1):
    wid = lax.axis_index("s") * _NC + lax.axis_index("c")
    sidx = (sidx0, sidx1)
    ridx = (ridx0, ridx1)
    a = (a0, a1)
    b = (b0, b1)
    sem = (sem0, sem1)

    pltpu.sync_copy(box_hbm, box_v)
    bx = box_v[0, :]
    by = box_v[1, :]
    bz = box_v[2, :]
    ibx = box_v[3, :]
    iby = box_v[4, :]
    ibz = box_v[5, :]

    iota = lax.iota(jnp.int32, _L)
    k0 = jnp.zeros((_L,), jnp.int32)
    k1 = jnp.full((_L,), 1, jnp.int32)
    k2 = jnp.full((_L,), 2, jnp.int32)
    mrne = jnp.full((_L,), _RNE_MAGIC, jnp.float32)
    half = jnp.full((_L,), 0.5, jnp.float32)
    c15 = jnp.full((_L,), 1.5, jnp.float32)
    magic = jnp.full((_L,), _RSQRT_MAGIC, jnp.int32)

    def prefetch(c, g):
        row0 = _CR * c
        pltpu.sync_copy(s_hbm.at[pl.ds(row0, _CR)], sidx[g])
        pltpu.sync_copy(r_hbm.at[pl.ds(row0, _CR)], ridx[g])
        for j in range(_CR):
            pltpu.async_copy(tab_hbm.at[sidx[g].at[j]],
                             a[g].at[pl.ds(j * _ROW, _ROW)], sem[g])
            pltpu.async_copy(tab_hbm.at[ridx[g].at[j]],
                             b[g].at[pl.ds(j * _ROW, _ROW)], sem[g])

    def drain(g):
        for j in range(_CR):
            pltpu.make_async_copy(tab_hbm.at[sidx[g].at[j]],
                                  a[g].at[pl.ds(j * _ROW, _ROW)], sem[g]).wait()
            pltpu.make_async_copy(tab_hbm.at[ridx[g].at[j]],
                                  b[g].at[pl.ds(j * _ROW, _ROW)], sem[g]).wait()

    @pl.when(wid < nchunks)
    def _():
        prefetch(wid, 0)

    def process(c, g, a_v, b_v):
        @pl.when(c < nchunks)
        def _():
            drain(g)

            def step(i, carry2):
                e16 = i * _L + iota
                ax = plsc.load_gather(a_v, [e16, k0])
                ay = plsc.load_gather(a_v, [e16, k1])
                az = plsc.load_gather(a_v, [e16, k2])
                qx = plsc.load_gather(b_v, [e16, k0])
                qy = plsc.load_gather(b_v, [e16, k1])
                qz = plsc.load_gather(b_v, [e16, k2])
                dx = qx - ax
                dy = qy - ay
                dz = qz - az
                nx = (dx * ibx + mrne) - mrne
                ny = (dy * iby + mrne) - mrne
                nz = (dz * ibz + mrne) - mrne
                dx = dx - nx * bx
                dy = dy - ny * by
                dz = dz - nz * bz
                s = dx * dx + dy * dy + dz * dz
                yi = magic - lax.shift_right_logical(plsc.bitcast(s, jnp.int32), 1)
                y = plsc.bitcast(yi, jnp.float32)
                hs = s * half
                y = y * (c15 - hs * y * y)
                y = y * (c15 - hs * y * y)
                y = y * (c15 - hs * y * y)
                rr = s * y
                sl = pl.ds(i * _L, _L)
                px_v[sl] = dx
                py_v[sl] = dy
                pz_v[sl] = dz
                rr_v[sl] = rr
                return carry2

            lax.fori_loop(0, _CHUNK // _L, step, 0, unroll=2)

            base = c * _CHUNK
            pltpu.sync_copy(px_v, out_x.at[pl.ds(base, _CHUNK)])
            pltpu.sync_copy(py_v, out_y.at[pl.ds(base, _CHUNK)])
            pltpu.sync_copy(pz_v, out_z.at[pl.ds(base, _CHUNK)])
            pltpu.sync_copy(rr_v, out_r.at[pl.ds(base, _CHUNK)])

    def pair_body(u, carry):
        for g in (0, 1):
            t = 2 * u + g
            c = wid + _NW * t
            cn = c + _NW

            @pl.when(cn < nchunks)
            def _():
                prefetch(cn, 1 - g)

            process(c, g, a[g], b[g])
        return carry

    lax.fori_loop(0, (trips + 1) // 2, pair_body, 0)


def kernel(coords, boxsize, senders, receivers):
    n_edges = senders.shape[0]
    assert n_edges % _CHUNK == 0
    nchunks = n_edges // _CHUNK
    trips = (nchunks + _NW - 1) // _NW

    s32 = senders.astype(jnp.int32).reshape(n_edges // _ROW, _ROW)
    r32 = receivers.astype(jnp.int32).reshape(n_edges // _ROW, _ROW)
    tab = jnp.pad(coords.astype(jnp.float32), ((0, 0), (0, 13)))
    box3 = boxsize.astype(jnp.float32).reshape(3)
    rows = [box3[0], box3[1], box3[2], 1.0 / box3[0], 1.0 / box3[1],
            1.0 / box3[2], jnp.float32(0.0), jnp.float32(0.0)]
    box_tab = jnp.stack([jnp.full((_L,), v, jnp.float32) for v in rows])

    mesh = plsc.VectorSubcoreMesh(core_axis_name="c", subcore_axis_name="s")
    f = functools.partial(
        pl.kernel,
        mesh=mesh,
        compiler_params=pltpu.CompilerParams(
            needs_layout_passes=False, use_tc_tiling_on_sc=False),
        out_type=[
            jax.ShapeDtypeStruct((n_edges,), jnp.float32),
            jax.ShapeDtypeStruct((n_edges,), jnp.float32),
            jax.ShapeDtypeStruct((n_edges,), jnp.float32),
            jax.ShapeDtypeStruct((n_edges,), jnp.float32),
        ],
        scratch_types=[
            pltpu.VMEM((_CR, _ROW), jnp.int32),
            pltpu.VMEM((_CR, _ROW), jnp.int32),
            pltpu.VMEM((_CHUNK, 16), jnp.float32),
            pltpu.VMEM((_CHUNK, 16), jnp.float32),
            pltpu.VMEM((_CR, _ROW), jnp.int32),
            pltpu.VMEM((_CR, _ROW), jnp.int32),
            pltpu.VMEM((_CHUNK, 16), jnp.float32),
            pltpu.VMEM((_CHUNK, 16), jnp.float32),
            pltpu.VMEM((_CHUNK,), jnp.float32),
            pltpu.VMEM((_CHUNK,), jnp.float32),
            pltpu.VMEM((_CHUNK,), jnp.float32),
            pltpu.VMEM((_CHUNK,), jnp.float32),
            pltpu.VMEM((8, _L), jnp.float32),
            pltpu.SemaphoreType.DMA,
            pltpu.SemaphoreType.DMA,
        ],
    )(functools.partial(_sc_body, nchunks, trips))
    px, py, pz, rr = f(s32, r32, tab, box_tab)
    rx = jnp.stack([px, py, pz], axis=1)
    return (rr.reshape(n_edges, 1), rx)
